# baseline (speedup 1.0000x reference)
import functools

import jax
import jax.numpy as jnp
from jax.experimental import pallas as pl
from jax.experimental.pallas import tpu as pltpu


def _round_up(x, m):
    return ((x + m - 1) // m) * m


_CHUNK = 2048


def _mlp_kernel(state_ref, action_ref, w1s_ref, w1a_ref, b_ref, w2_ref, w3_ref,
                out_ref, w1sb_ref, w1ab_ref, w2b_ref, w3b_ref):
    Ds = out_ref.shape[-1]

    @pl.when(pl.program_id(0) == 0)
    def _cast_weights():
        w1sb_ref[...] = w1s_ref[...].astype(jnp.bfloat16)
        w1ab_ref[...] = w1a_ref[...].astype(jnp.bfloat16)
        w2b_ref[...] = w2_ref[...].astype(jnp.bfloat16)
        w3b_ref[...] = w3_ref[...].astype(jnp.bfloat16)

    b = b_ref[...]
    bb = out_ref.shape[0]
    for c0 in range(0, bb, _CHUNK):
        sl = pl.ds(c0, _CHUNK)
        s = state_ref[sl, :].astype(jnp.bfloat16)
        a = action_ref[sl, :].astype(jnp.bfloat16)
        h1 = (jnp.dot(s, w1sb_ref[...], preferred_element_type=jnp.float32)
              + jnp.dot(a, w1ab_ref[...], preferred_element_type=jnp.float32)
              + b[0:1, :])
        h1 = jnp.maximum(h1, 0.0).astype(jnp.bfloat16)
        h2 = jnp.dot(h1, w2b_ref[...],
                     preferred_element_type=jnp.float32) + b[1:2, :]
        h2 = jnp.maximum(h2, 0.0).astype(jnp.bfloat16)
        out = jnp.dot(h2, w3b_ref[...],
                      preferred_element_type=jnp.float32) + b[2:3, :Ds]
        out_ref[sl, :] = out.astype(out_ref.dtype)


@functools.partial(jax.jit, static_argnames=("block_b",))
def _run(state, action, w1s, w1a, b_packed, w2p, w3p, *, block_b=4096):
    Ds = state.shape[-1]
    Da = action.shape[-1]
    batch_shape = state.shape[:-1]

    s2 = state.reshape(-1, Ds)
    a2 = action.reshape(-1, Da)
    B = s2.shape[0]

    H1 = w1s.shape[1]
    H2 = w2p.shape[1]

    bb = _round_up(min(block_b, _round_up(B, 8)), 8)
    Bp = _round_up(B, bb)
    if Bp != B:
        s2 = jnp.pad(s2, ((0, Bp - B), (0, 0)))
        a2 = jnp.pad(a2, ((0, Bp - B), (0, 0)))

    grid = (Bp // bb,)

    def full_spec(arr):
        return pl.BlockSpec(arr.shape, lambda i: (0, 0))

    row_map = lambda i: (i, 0)

    cost = pl.CostEstimate(
        flops=2 * Bp * ((Ds + Da) * H1 + H1 * H2 + H2 * Ds),
        transcendentals=0,
        bytes_accessed=4 * Bp * (Ds + Da + Ds),
    )

    out = pl.pallas_call(
        _mlp_kernel,
        out_shape=jax.ShapeDtypeStruct((Bp, Ds), state.dtype),
        grid=grid,
        in_specs=[
            pl.BlockSpec((bb, Ds), row_map),
            pl.BlockSpec((bb, Da), row_map),
            full_spec(w1s), full_spec(w1a), full_spec(b_packed),
            full_spec(w2p), full_spec(w3p),
        ],
        out_specs=pl.BlockSpec((bb, Ds), row_map),
        scratch_shapes=[
            pltpu.VMEM((Ds, H1), jnp.bfloat16),
            pltpu.VMEM((Da, H1), jnp.bfloat16),
            pltpu.VMEM((H1, H2), jnp.bfloat16),
            pltpu.VMEM((H2, Ds), jnp.bfloat16),
        ],
        compiler_params=pltpu.CompilerParams(
            dimension_semantics=("arbitrary",)),
        cost_estimate=cost,
    )(s2, a2, w1s, w1a, b_packed, w2p, w3p)

    out = out[:B]
    return out.reshape(*batch_shape, Ds)


def kernel(state, action, w1s, w1a, b_packed, w2p, w3p):
    return _run(state, action, w1s, w1a, b_packed, w2p, w3p)


# speedup vs baseline: 1.0338x; 1.0338x over previous
import functools

import jax
import jax.numpy as jnp
from jax.experimental import pallas as pl
from jax.experimental.pallas import tpu as pltpu


def _round_up(x, m):
    return ((x + m - 1) // m) * m


def _mlp_kernel(state_ref, action_ref, w1s_ref, w1a_ref, b_ref, w2_ref, w3_ref,
                out_ref, w1sb_ref, w1ab_ref, w2b_ref, w3b_ref):
    Ds = out_ref.shape[-1]

    @pl.when(pl.program_id(0) == 0)
    def _cast_weights():
        w1sb_ref[...] = w1s_ref[...].astype(jnp.bfloat16)
        w1ab_ref[...] = w1a_ref[...].astype(jnp.bfloat16)
        w2b_ref[...] = w2_ref[...].astype(jnp.bfloat16)
        w3b_ref[...] = w3_ref[...].astype(jnp.bfloat16)

    b = b_ref[...]
    s = state_ref[...].astype(jnp.bfloat16)
    a = action_ref[...].astype(jnp.bfloat16)

    h1 = (jnp.dot(s, w1sb_ref[...], preferred_element_type=jnp.float32)
          + jnp.dot(a, w1ab_ref[...], preferred_element_type=jnp.float32)
          + b[0:1, :])
    h1 = jnp.maximum(h1, 0.0).astype(jnp.bfloat16)

    h2 = jnp.dot(h1, w2b_ref[...],
                 preferred_element_type=jnp.float32) + b[1:2, :]
    h2 = jnp.maximum(h2, 0.0).astype(jnp.bfloat16)

    out = jnp.dot(h2, w3b_ref[...],
                  preferred_element_type=jnp.float32) + b[2:3, :Ds]
    out_ref[...] = out.astype(out_ref.dtype)


@functools.partial(jax.jit, static_argnames=("block_b",))
def _run(state, action, w1s, w1a, b_packed, w2p, w3p, *, block_b=4096):
    Ds = state.shape[-1]
    Da = action.shape[-1]
    batch_shape = state.shape[:-1]

    s2 = state.reshape(-1, Ds)
    a2 = action.reshape(-1, Da)
    B = s2.shape[0]

    H1 = w1s.shape[1]
    H2 = w2p.shape[1]

    bb = _round_up(min(block_b, _round_up(B, 8)), 8)
    Bp = _round_up(B, bb)
    if Bp != B:
        s2 = jnp.pad(s2, ((0, Bp - B), (0, 0)))
        a2 = jnp.pad(a2, ((0, Bp - B), (0, 0)))

    grid = (Bp // bb,)

    def full_spec(arr):
        return pl.BlockSpec(arr.shape, lambda i: (0, 0))

    row_map = lambda i: (i, 0)

    cost = pl.CostEstimate(
        flops=2 * Bp * ((Ds + Da) * H1 + H1 * H2 + H2 * Ds),
        transcendentals=0,
        bytes_accessed=4 * Bp * (Ds + Da + Ds),
    )

    out = pl.pallas_call(
        _mlp_kernel,
        out_shape=jax.ShapeDtypeStruct((Bp, Ds), state.dtype),
        grid=grid,
        in_specs=[
            pl.BlockSpec((bb, Ds), row_map),
            pl.BlockSpec((bb, Da), row_map),
            full_spec(w1s), full_spec(w1a), full_spec(b_packed),
            full_spec(w2p), full_spec(w3p),
        ],
        out_specs=pl.BlockSpec((bb, Ds), row_map),
        scratch_shapes=[
            pltpu.VMEM((Ds, H1), jnp.bfloat16),
            pltpu.VMEM((Da, H1), jnp.bfloat16),
            pltpu.VMEM((H1, H2), jnp.bfloat16),
            pltpu.VMEM((H2, Ds), jnp.bfloat16),
        ],
        compiler_params=pltpu.CompilerParams(
            dimension_semantics=("arbitrary",)),
        cost_estimate=cost,
    )(s2, a2, w1s, w1a, b_packed, w2p, w3p)

    out = out[:B]
    return out.reshape(*batch_shape, Ds)


def kernel(state, action, w1s, w1a, b_packed, w2p, w3p):
    return _run(state, action, w1s, w1a, b_packed, w2p, w3p)
